# Initial kernel scaffold; baseline (speedup 1.0000x reference)
import jax
import jax.numpy as jnp
from jax.experimental import pallas as pl


def kernel(node_type, node_state_type, edge_index, edge_type, batch, non_edge_index, node_table, state_table, edge_table, virt_table, layer_params):
    raise NotImplementedError("write your pallas kernel here")


# speedup vs baseline: 8.1136x; 8.1136x over previous
import functools

import jax
import jax.numpy as jnp
from jax import lax
from jax.experimental import pallas as pl
from jax.experimental.pallas import tpu as pltpu
from jax.experimental.pallas import tpu_sc as plsc

N = 10000
E = 320000
B = 64
NE = 20000
D = 128
H = 2
L = 3
Nn = N + B
NnP = 10240
EP = 323584
NEP = 20480
EPS_GEN = 1e-7
EPS_LN = 1e-5
NBLK = 512
EBLK = 512
NW = 32
CTW = EP // (NW * 128)
CTS = EP // (16 * 128)
CTN = NEP // (NW * 128)
BIG = 3.0e38
ISD = 1.0 / (D ** 0.5)

_PC = pl.pallas_call


def _dotT(a, b):
    return lax.dot_general(a, b, (((0,), (0,)), ((), ())),
                           preferred_element_type=jnp.float32)


def _bs(shape, imap):
    return pl.BlockSpec(shape, imap)


def _colsum(m):
    R, C = m.shape
    g = R // NBLK

    def body(m_ref, o_ref):
        i = pl.program_id(0)

        @pl.when(i == 0)
        def _():
            o_ref[...] = jnp.zeros_like(o_ref)
        o_ref[...] += _dotT(m_ref[...], jnp.ones((NBLK, 1), jnp.float32))

    return _PC(body, grid=(g,),
               in_specs=[_bs((NBLK, C), lambda i: (i, 0))],
               out_specs=_bs((C, 1), lambda i: (0, 0)),
               out_shape=jax.ShapeDtypeStruct((C, 1), jnp.float32))(m)


def _embed(oh, tbl):
    def body(oh_ref, t_ref, o_ref):
        o_ref[...] = jnp.dot(oh_ref[...], t_ref[...],
                             preferred_element_type=jnp.float32)

    return _PC(body, grid=(NnP // NBLK,),
               in_specs=[_bs((NBLK, 32), lambda i: (i, 0)),
                         _bs((32, D), lambda i: (0, 0))],
               out_specs=_bs((NBLK, D), lambda i: (i, 0)),
               out_shape=jax.ShapeDtypeStruct((NnP, D), jnp.float32))(oh, tbl)


def _lnstats(x, oh):
    g = NnP // NBLK

    def body(x_ref, oh_ref, o_ref, sx, sxx, cnt):
        i = pl.program_id(0)

        @pl.when(i == 0)
        def _():
            sx[...] = jnp.zeros_like(sx)
            sxx[...] = jnp.zeros_like(sxx)
            cnt[...] = jnp.zeros_like(cnt)
        xb = x_ref[...]
        ohb = oh_ref[...]
        sx[...] += _dotT(ohb, xb)
        sxx[...] += _dotT(ohb, xb * xb)
        cnt[...] += _dotT(ohb, jnp.ones((NBLK, 1), jnp.float32))

        @pl.when(i == g - 1)
        def _():
            cf = jnp.maximum(cnt[...] * float(D), 1.0)
            inv = 1.0 / cf
            mean = jnp.sum(sx[...], axis=1, keepdims=True) * inv
            exx = jnp.sum(sxx[...], axis=1, keepdims=True) * inv
            var = exx - mean * mean
            rstd = lax.rsqrt(var + EPS_LN)
            o_ref[...] = jnp.concatenate(
                [mean, rstd, jnp.zeros((B, 6), jnp.float32)], axis=1)

    return _PC(body, grid=(g,),
               in_specs=[_bs((NBLK, D), lambda i: (i, 0)),
                         _bs((NBLK, B), lambda i: (i, 0))],
               out_specs=_bs((B, 8), lambda i: (0, 0)),
               out_shape=jax.ShapeDtypeStruct((B, 8), jnp.float32),
               scratch_shapes=[pltpu.VMEM((B, D), jnp.float32),
                               pltpu.VMEM((B, D), jnp.float32),
                               pltpu.VMEM((B, 1), jnp.float32)])(x, oh)


def _norm(x, oh, st):
    def body(x_ref, oh_ref, st_ref, o_ref):
        mr = jnp.dot(oh_ref[...], st_ref[...][:, 0:2],
                     preferred_element_type=jnp.float32)
        o_ref[...] = (x_ref[...] - mr[:, 0:1]) * mr[:, 1:2]

    return _PC(body, grid=(NnP // NBLK,),
               in_specs=[_bs((NBLK, D), lambda i: (i, 0)),
                         _bs((NBLK, B), lambda i: (i, 0)),
                         _bs((B, 8), lambda i: (0, 0))],
               out_specs=_bs((NBLK, D), lambda i: (i, 0)),
               out_shape=jax.ShapeDtypeStruct((NnP, D), jnp.float32))(x, oh, st)


def _msg(gs, ohet, et8):
    def body(g_ref, oh_ref, t_ref, o_ref):
        ohb = oh_ref[...]
        t = jnp.dot(ohb, t_ref[...], preferred_element_type=jnp.float32)
        msk = jnp.sum(ohb, axis=1, keepdims=True)
        o_ref[...] = (jax.nn.relu(g_ref[...] + t) + EPS_GEN) * msk

    return _PC(body, grid=(EP // EBLK,),
               in_specs=[_bs((EBLK, D), lambda i: (i, 0)),
                         _bs((EBLK, 8), lambda i: (i, 0)),
                         _bs((8, D), lambda i: (0, 0))],
               out_specs=_bs((EBLK, D), lambda i: (i, 0)),
               out_shape=jax.ShapeDtypeStruct((EP, D), jnp.float32))(gs, ohet, et8)


def _uvsum(xn, ohn, t4):
    g = NnP // NBLK

    def body(x_ref, oh_ref, t4_ref, o_ref):
        i = pl.program_id(0)

        @pl.when(i == 0)
        def _():
            o_ref[...] = jnp.zeros_like(o_ref)
        m = jax.nn.relu(x_ref[...] + t4_ref[...]) + EPS_GEN
        o_ref[...] += _dotT(oh_ref[...], m)

    return _PC(body, grid=(g,),
               in_specs=[_bs((NBLK, D), lambda i: (i, 0)),
                         _bs((NBLK, B), lambda i: (i, 0)),
                         _bs((1, D), lambda i: (0, 0))],
               out_specs=_bs((B, D), lambda i: (0, 0)),
               out_shape=jax.ShapeDtypeStruct((B, D), jnp.float32))(xn, ohn, t4)


def _aggqkvs(a0, a1, xn, ohn, ohv, uv, kvirt, t4, t5, genw, genb, wa, wb, bc):
    def body(a0_ref, a1_ref, x_ref, ohn_ref, ohv_ref, uv_ref, kv_ref, t4_ref,
             t5_ref, gw_ref, gb_ref, wa_ref, wb_ref, bc_ref, q_ref, kvo_ref,
             sk_ref):
        xb = x_ref[...]
        ohn_b = ohn_ref[...]
        is_node = jnp.sum(ohn_b, axis=1, keepdims=True)
        m_self = jax.nn.relu(xb + t5_ref[...]) + EPS_GEN
        xv = jnp.dot(ohn_b, kv_ref[...], preferred_element_type=jnp.float32)
        m_vu = (jax.nn.relu(xv + t4_ref[...]) + EPS_GEN) * is_node
        m_uv = jnp.dot(ohv_ref[...], uv_ref[...],
                       preferred_element_type=jnp.float32)
        agg = a0_ref[...] + a1_ref[...] + m_self + m_vu + m_uv
        gen = jnp.dot(agg + xb, gw_ref[...],
                      preferred_element_type=jnp.float32) + gb_ref[...]
        qkvs = (jnp.dot(xb, wa_ref[...], preferred_element_type=jnp.float32)
                + jnp.dot(gen, wb_ref[...], preferred_element_type=jnp.float32)
                + bc_ref[...])
        q_ref[...] = qkvs[:, 0:256]
        kvo_ref[...] = qkvs[:, 256:768]
        sk_ref[...] = qkvs[:, 768:1024]

    return _PC(body, grid=(NnP // NBLK,),
               in_specs=[_bs((NBLK, D), lambda i: (i, 0)),
                         _bs((NBLK, D), lambda i: (i, 0)),
                         _bs((NBLK, D), lambda i: (i, 0)),
                         _bs((NBLK, B), lambda i: (i, 0)),
                         _bs((NBLK, B), lambda i: (i, 0)),
                         _bs((B, D), lambda i: (0, 0)),
                         _bs((B, D), lambda i: (0, 0)),
                         _bs((1, D), lambda i: (0, 0)),
                         _bs((1, D), lambda i: (0, 0)),
                         _bs((D, D), lambda i: (0, 0)),
                         _bs((1, D), lambda i: (0, 0)),
                         _bs((D, 1024), lambda i: (0, 0)),
                         _bs((D, 1024), lambda i: (0, 0)),
                         _bs((1, 1024), lambda i: (0, 0))],
               out_specs=[_bs((NBLK, 256), lambda i: (i, 0)),
                          _bs((NBLK, 512), lambda i: (i, 0)),
                          _bs((NBLK, 256), lambda i: (i, 0))],
               out_shape=[jax.ShapeDtypeStruct((NnP, 256), jnp.float32),
                          jax.ShapeDtypeStruct((NnP, 512), jnp.float32),
                          jax.ShapeDtypeStruct((NnP, 256), jnp.float32)],
               )(a0, a1, xn, ohn, ohv, uv, kvirt, t4, t5, genw, genb, wa, wb,
                 bc)


def _hdot(a, b):
    p0 = jnp.sum(a[:, 0:128] * b[:, 0:128], axis=1, keepdims=True)
    p1 = jnp.sum(a[:, 128:256] * b[:, 128:256], axis=1, keepdims=True)
    return jnp.concatenate([p0, p1], axis=1) * ISD


def _structalpha(q, kv, ohn, vq, vkv, eet):
    g = NnP // NBLK

    def body(q_ref, kv_ref, ohn_ref, vq_ref, vkv_ref, ee_ref, sa_ref, bnd_ref):
        i = pl.program_id(0)
        qb = q_ref[...]
        kb = kv_ref[...][:, 0:256]
        ohn_b = ohn_ref[...]
        is_node = jnp.sum(ohn_b, axis=1, keepdims=True)
        ee4 = ee_ref[...][4:5, :]
        ee5 = ee_ref[...][5:6, :]
        a_self = _hdot(qb, kb + ee5)
        kvu = jnp.dot(ohn_b, vkv_ref[...][:, 0:256],
                      preferred_element_type=jnp.float32) + ee4
        a_vu = _hdot(qb, kvu)
        quv = jnp.dot(ohn_b, vq_ref[...],
                      preferred_element_type=jnp.float32)
        a_uv = _hdot(quv, kb + ee4)
        sa_ref[...] = jnp.concatenate(
            [a_self, a_vu, a_uv, jnp.zeros((NBLK, 2), jnp.float32)], axis=1)
        lo = jnp.min(jnp.where(is_node > 0.0, a_uv, BIG))
        nhi = jnp.min(jnp.where(is_node > 0.0, -a_uv, BIG))
        row = jnp.concatenate(
            [jnp.full((1, 1), lo), jnp.full((1, 1), nhi),
             jnp.full((1, 6), BIG)], axis=1)

        @pl.when(i == 0)
        def _():
            bnd_ref[...] = jnp.full((1, 8), BIG)
        bnd_ref[...] = jnp.minimum(bnd_ref[...], row)

    return _PC(body, grid=(g,),
               in_specs=[_bs((NBLK, 256), lambda i: (i, 0)),
                         _bs((NBLK, 512), lambda i: (i, 0)),
                         _bs((NBLK, B), lambda i: (i, 0)),
                         _bs((B, 256), lambda i: (0, 0)),
                         _bs((B, 512), lambda i: (0, 0)),
                         _bs((8, 256), lambda i: (0, 0))],
               out_specs=[_bs((NBLK, 8), lambda i: (i, 0)),
                          _bs((1, 8), lambda i: (0, 0))],
               out_shape=[jax.ShapeDtypeStruct((NnP, 8), jnp.float32),
                          jax.ShapeDtypeStruct((1, 8), jnp.float32)],
               )(q, kv, ohn, vq, vkv, eet)


def _alpha(gq, gkv, ohet, eet):
    g = EP // EBLK

    def body(gq_ref, gkv_ref, oh_ref, ee_ref, ae_ref, bnd_ref):
        i = pl.program_id(0)
        ohb = oh_ref[...]
        msk = jnp.sum(ohb, axis=1, keepdims=True)
        gk = gkv_ref[...][:, 0:256] + jnp.dot(
            ohb, ee_ref[...], preferred_element_type=jnp.float32)
        a = _hdot(gq_ref[...], gk)
        ae_ref[...] = jnp.concatenate(
            [a * msk, jnp.zeros((EBLK, 6), jnp.float32)], axis=1)
        lo = jnp.min(jnp.where(msk > 0.0, a, BIG))
        nhi = jnp.min(jnp.where(msk > 0.0, -a, BIG))
        row = jnp.concatenate(
            [jnp.full((1, 1), lo), jnp.full((1, 1), nhi),
             jnp.full((1, 6), BIG)], axis=1)

        @pl.when(i == 0)
        def _():
            bnd_ref[...] = jnp.full((1, 8), BIG)
        bnd_ref[...] = jnp.minimum(bnd_ref[...], row)

    return _PC(body, grid=(g,),
               in_specs=[_bs((EBLK, 256), lambda i: (i, 0)),
                         _bs((EBLK, 512), lambda i: (i, 0)),
                         _bs((EBLK, 8), lambda i: (i, 0)),
                         _bs((8, 256), lambda i: (0, 0))],
               out_specs=[_bs((EBLK, 8), lambda i: (i, 0)),
                          _bs((1, 8), lambda i: (0, 0))],
               out_shape=[jax.ShapeDtypeStruct((EP, 8), jnp.float32),
                          jax.ShapeDtypeStruct((1, 8), jnp.float32)],
               )(gq, gkv, ohet, eet)


def _abw(bnd_e, bnd_s):
    a_lo = jnp.minimum(bnd_e[0:1, 0:1], bnd_s[0:1, 0:1])
    a_hi = jnp.maximum(-bnd_e[0:1, 1:2], -bnd_s[0:1, 1:2])
    w = (a_hi - a_lo) * (1.0 / 16.0)
    winv = 1.0 / jnp.maximum(w, 1e-30)
    return a_lo, w, winv


def _lvl16(a, a_lo, winv):
    lvl = jnp.clip(jnp.floor((a - a_lo) * winv), 0.0, 15.0)
    R = a.shape[0]
    io = lax.broadcasted_iota(jnp.int32, (R, 16), 1).astype(jnp.float32)
    return jnp.where(io == lvl, 1.0, 0.0)


def _bucket(ae, ohet, bnd_e, bnd_s):
    def body(ae_ref, oh_ref, be_ref, bs_ref, o_ref):
        a_lo, _, winv = _abw(be_ref[...], bs_ref[...])
        msk = jnp.sum(oh_ref[...], axis=1, keepdims=True)
        aeb = ae_ref[...]
        o0 = _lvl16(aeb[:, 0:1], a_lo, winv) * msk
        o1 = _lvl16(aeb[:, 1:2], a_lo, winv) * msk
        o_ref[...] = jnp.concatenate(
            [o0, o1, jnp.zeros((EBLK, 96), jnp.float32)], axis=1)

    return _PC(body, grid=(EP // EBLK,),
               in_specs=[_bs((EBLK, 8), lambda i: (i, 0)),
                         _bs((EBLK, 8), lambda i: (i, 0)),
                         _bs((1, 8), lambda i: (0, 0)),
                         _bs((1, 8), lambda i: (0, 0))],
               out_specs=_bs((EBLK, 128), lambda i: (i, 0)),
               out_shape=jax.ShapeDtypeStruct((EP, 128), jnp.float32),
               )(ae, ohet, bnd_e, bnd_s)


def _cntu(sa, ohn, bnd_e, bnd_s):
    g = NnP // NBLK

    def body(sa_ref, ohn_ref, be_ref, bs_ref, o_ref):
        i = pl.program_id(0)

        @pl.when(i == 0)
        def _():
            o_ref[...] = jnp.zeros_like(o_ref)
        a_lo, _, winv = _abw(be_ref[...], bs_ref[...])
        ohn_b = ohn_ref[...]
        is_node = jnp.sum(ohn_b, axis=1, keepdims=True)
        sab = sa_ref[...]
        o0 = _lvl16(sab[:, 4:5], a_lo, winv) * is_node
        o1 = _lvl16(sab[:, 5:6], a_lo, winv) * is_node
        o_ref[...] += _dotT(ohn_b, jnp.concatenate([o0, o1], axis=1))

    return _PC(body, grid=(g,),
               in_specs=[_bs((NBLK, 8), lambda i: (i, 0)),
                         _bs((NBLK, B), lambda i: (i, 0)),
                         _bs((1, 8), lambda i: (0, 0)),
                         _bs((1, 8), lambda i: (0, 0))],
               out_specs=_bs((B, 32), lambda i: (0, 0)),
               out_shape=jax.ShapeDtypeStruct((B, 32), jnp.float32),
               )(sa, ohn, bnd_e, bnd_s)


def _cmax(c0, c1, ohv, ohn, cntu, sa, bnd_e, bnd_s):
    def body(c0_ref, c1_ref, ohv_ref, ohn_ref, cu_ref, sa_ref, be_ref, bs_ref,
             o_ref):
        a_lo, w, _ = _abw(be_ref[...], bs_ref[...])
        cnt = c0_ref[...][:, 0:32] + c1_ref[...][:, 0:32] + jnp.dot(
            ohv_ref[...], cu_ref[...], preferred_element_type=jnp.float32)
        is_node = jnp.sum(ohn_ref[...], axis=1, keepdims=True)
        sab = sa_ref[...]
        cols = []
        for h in range(H):
            ch = cnt[:, 16 * h:16 * h + 16]
            io = lax.broadcasted_iota(jnp.int32, (NBLK, 16), 1).astype(jnp.float32)
            top = jnp.max(jnp.where(ch > 0.0, io, -1.0), axis=1,
                          keepdims=True)
            cb = jnp.where(top >= 0.0, a_lo + w * (top + 1.0), -BIG)
            cb = jnp.maximum(cb, sab[:, h:h + 1])
            avu = jnp.where(is_node > 0.0, sab[:, 2 + h:3 + h], -BIG)
            cols.append(jnp.maximum(cb, avu))
        o_ref[...] = jnp.concatenate(
            cols + [jnp.zeros((NBLK, 126), jnp.float32)], axis=1)

    return _PC(body, grid=(NnP // NBLK,),
               in_specs=[_bs((NBLK, 128), lambda i: (i, 0)),
                         _bs((NBLK, 128), lambda i: (i, 0)),
                         _bs((NBLK, B), lambda i: (i, 0)),
                         _bs((NBLK, B), lambda i: (i, 0)),
                         _bs((B, 32), lambda i: (0, 0)),
                         _bs((NBLK, 8), lambda i: (i, 0)),
                         _bs((1, 8), lambda i: (0, 0)),
                         _bs((1, 8), lambda i: (0, 0))],
               out_specs=_bs((NBLK, 128), lambda i: (i, 0)),
               out_shape=jax.ShapeDtypeStruct((NnP, 128), jnp.float32),
               )(c0, c1, ohv, ohn, cntu, sa, bnd_e, bnd_s)


def _structnum(c, sa, kv, ohn, cv, vkv, eet):
    g = NnP // NBLK

    def body(c_ref, sa_ref, kv_ref, ohn_ref, cv_ref, vkv_ref, ee_ref,
             nums_ref, dens_ref, uvn_ref, uvd_ref):
        i = pl.program_id(0)

        @pl.when(i == 0)
        def _():
            uvn_ref[...] = jnp.zeros_like(uvn_ref)
            uvd_ref[...] = jnp.zeros_like(uvd_ref)
        cb = c_ref[...]
        sab = sa_ref[...]
        vb = kv_ref[...][:, 256:512]
        ohn_b = ohn_ref[...]
        is_node = jnp.sum(ohn_b, axis=1, keepdims=True)
        ee4 = ee_ref[...][4:5, :]
        ee5 = ee_ref[...][5:6, :]
        vself = vb + ee5
        vvu = jnp.dot(ohn_b, vkv_ref[...][:, 256:512],
                      preferred_element_type=jnp.float32) + ee4
        cuv = jnp.dot(ohn_b, cv_ref[...][:, 0:2],
                      preferred_element_type=jnp.float32)
        vuv = vb + ee4
        nums, dens, uvn, uvd = [], [], [], []
        for h in range(H):
            ch = cb[:, h:h + 1]
            ex_s = jnp.exp(sab[:, h:h + 1] - ch)
            ex_v = jnp.exp(sab[:, 2 + h:3 + h] - ch) * is_node
            ex_u = jnp.exp(sab[:, 4 + h:5 + h] - cuv[:, h:h + 1]) * is_node
            sl = slice(128 * h, 128 * h + 128)
            nums.append(ex_s * vself[:, sl] + ex_v * vvu[:, sl])
            dens.append(ex_s + ex_v)
            uvn.append(ex_u * vuv[:, sl])
            uvd.append(ex_u)
        nums_ref[...] = jnp.concatenate(nums, axis=1)
        dens_ref[...] = jnp.concatenate(
            dens + [jnp.zeros((NBLK, 6), jnp.float32)], axis=1)
        uvn_ref[...] += _dotT(ohn_b, jnp.concatenate(uvn, axis=1))
        uvd_ref[...] += _dotT(ohn_b, jnp.concatenate(
            uvd + [jnp.zeros((NBLK, 6), jnp.float32)], axis=1))

    return _PC(body, grid=(g,),
               in_specs=[_bs((NBLK, 128), lambda i: (i, 0)),
                         _bs((NBLK, 8), lambda i: (i, 0)),
                         _bs((NBLK, 512), lambda i: (i, 0)),
                         _bs((NBLK, B), lambda i: (i, 0)),
                         _bs((B, 128), lambda i: (0, 0)),
                         _bs((B, 512), lambda i: (0, 0)),
                         _bs((8, 256), lambda i: (0, 0))],
               out_specs=[_bs((NBLK, 256), lambda i: (i, 0)),
                          _bs((NBLK, 8), lambda i: (i, 0)),
                          _bs((B, 256), lambda i: (0, 0)),
                          _bs((B, 8), lambda i: (0, 0))],
               out_shape=[jax.ShapeDtypeStruct((NnP, 256), jnp.float32),
                          jax.ShapeDtypeStruct((NnP, 8), jnp.float32),
                          jax.ShapeDtypeStruct((B, 256), jnp.float32),
                          jax.ShapeDtypeStruct((B, 8), jnp.float32)],
               )(c, sa, kv, ohn, cv, vkv, eet)


def _rbuild(ae, gkv, ohet, cg, eet):
    def body(ae_ref, gkv_ref, oh_ref, cg_ref, ee_ref, r0_ref, r1_ref,
             dn_ref):
        ohb = oh_ref[...]
        msk = jnp.sum(ohb, axis=1, keepdims=True)
        gv = gkv_ref[...][:, 256:512] + jnp.dot(
            ohb, ee_ref[...], preferred_element_type=jnp.float32)
        aeb = ae_ref[...]
        cgb = cg_ref[...]
        exs = []
        for h, r_ref in ((0, r0_ref), (1, r1_ref)):
            ex = jnp.exp(aeb[:, h:h + 1] - cgb[:, h:h + 1]) * msk
            exs.append(ex)
            r_ref[...] = ex * gv[:, 128 * h:128 * h + 128]
        dn_ref[...] = jnp.concatenate(
            exs + [jnp.zeros((EBLK, 126), jnp.float32)], axis=1)

    return _PC(body, grid=(EP // EBLK,),
               in_specs=[_bs((EBLK, 8), lambda i: (i, 0)),
                         _bs((EBLK, 512), lambda i: (i, 0)),
                         _bs((EBLK, 8), lambda i: (i, 0)),
                         _bs((EBLK, 128), lambda i: (i, 0)),
                         _bs((8, 256), lambda i: (0, 0))],
               out_specs=[_bs((EBLK, 128), lambda i: (i, 0)),
                          _bs((EBLK, 128), lambda i: (i, 0)),
                          _bs((EBLK, 128), lambda i: (i, 0))],
               out_shape=[jax.ShapeDtypeStruct((EP, 128), jnp.float32),
                          jax.ShapeDtypeStruct((EP, 128), jnp.float32),
                          jax.ShapeDtypeStruct((EP, 128), jnp.float32)],
               )(ae, gkv, ohet, cg, eet)


def _outk(nd0, nd1, dd0, dd1, nums, dens, uvn, uvd, ohv, sk, linw, linb):
    def body(nd0_ref, nd1_ref, dd0_ref, dd1_ref, nums_ref, dens_ref, uvn_ref,
             uvd_ref, ohv_ref, sk_ref, lw_ref, lb_ref, o_ref):
        ohv_b = ohv_ref[...]
        uvn_b = jnp.dot(ohv_b, uvn_ref[...], preferred_element_type=jnp.float32)
        uvd_b = jnp.dot(ohv_b, uvd_ref[...], preferred_element_type=jnp.float32)
        dd = dd0_ref[...][:, 0:2] + dd1_ref[...][:, 0:2]
        cols = []
        for h, nd_ref in ((0, nd0_ref), (1, nd1_ref)):
            sl = slice(128 * h, 128 * h + 128)
            num = nd_ref[...] + nums_ref[...][:, sl] + uvn_b[:, sl]
            den = (dd[:, h:h + 1] + dens_ref[...][:, h:h + 1]
                   + uvd_b[:, h:h + 1])
            cols.append(num / (den + 1e-16))
        out2 = jnp.concatenate(cols, axis=1) + sk_ref[...]
        o_ref[...] = jnp.dot(out2, lw_ref[...],
                             preferred_element_type=jnp.float32) + lb_ref[...]

    return _PC(body, grid=(NnP // NBLK,),
               in_specs=[_bs((NBLK, 128), lambda i: (i, 0)),
                         _bs((NBLK, 128), lambda i: (i, 0)),
                         _bs((NBLK, 128), lambda i: (i, 0)),
                         _bs((NBLK, 128), lambda i: (i, 0)),
                         _bs((NBLK, 256), lambda i: (i, 0)),
                         _bs((NBLK, 8), lambda i: (i, 0)),
                         _bs((B, 256), lambda i: (0, 0)),
                         _bs((B, 8), lambda i: (0, 0)),
                         _bs((NBLK, B), lambda i: (i, 0)),
                         _bs((NBLK, 256), lambda i: (i, 0)),
                         _bs((256, D), lambda i: (0, 0)),
                         _bs((1, D), lambda i: (0, 0))],
               out_specs=_bs((NBLK, D), lambda i: (i, 0)),
               out_shape=jax.ShapeDtypeStruct((NnP, D), jnp.float32),
               )(nd0, nd1, dd0, dd1, nums, dens, uvn, uvd, ohv, sk, linw,
                 linb)


def _ffn(lh, oh, st, x, w1, b1, w2, b2):
    def body(lh_ref, oh_ref, st_ref, x_ref, w1_ref, b1_ref, w2_ref, b2_ref,
             o_ref):
        mr = jnp.dot(oh_ref[...], st_ref[...][:, 0:2],
                     preferred_element_type=jnp.float32)
        hn = (lh_ref[...] - mr[:, 0:1]) * mr[:, 1:2]
        t = jnp.dot(hn, w1_ref[...],
                    preferred_element_type=jnp.float32) + b1_ref[...]
        t = jnp.maximum(t, 0.01 * t)
        h = jnp.dot(t, w2_ref[...],
                    preferred_element_type=jnp.float32) + b2_ref[...]
        o_ref[...] = x_ref[...] + h

    return _PC(body, grid=(NnP // NBLK,),
               in_specs=[_bs((NBLK, D), lambda i: (i, 0)),
                         _bs((NBLK, B), lambda i: (i, 0)),
                         _bs((B, 8), lambda i: (0, 0)),
                         _bs((NBLK, D), lambda i: (i, 0)),
                         _bs((D, 512), lambda i: (0, 0)),
                         _bs((1, 512), lambda i: (0, 0)),
                         _bs((512, D), lambda i: (0, 0)),
                         _bs((1, D), lambda i: (0, 0))],
               out_specs=_bs((NBLK, D), lambda i: (i, 0)),
               out_shape=jax.ShapeDtypeStruct((NnP, D), jnp.float32),
               )(lh, oh, st, x, w1, b1, w2, b2)


def _pool(x, ohn, cntn, xv):
    g = NnP // NBLK

    def body(x_ref, ohn_ref, cnt_ref, xv_ref, o_ref, acc):
        i = pl.program_id(0)

        @pl.when(i == 0)
        def _():
            acc[...] = jnp.zeros_like(acc)
        acc[...] += _dotT(ohn_ref[...], x_ref[...])

        @pl.when(i == g - 1)
        def _():
            o_ref[...] = acc[...] / jnp.maximum(cnt_ref[...], 1.0) + xv_ref[...]

    return _PC(body, grid=(g,),
               in_specs=[_bs((NBLK, D), lambda i: (i, 0)),
                         _bs((NBLK, B), lambda i: (i, 0)),
                         _bs((B, 1), lambda i: (0, 0)),
                         _bs((B, D), lambda i: (0, 0))],
               out_specs=_bs((B, D), lambda i: (0, 0)),
               out_shape=jax.ShapeDtypeStruct((B, D), jnp.float32),
               scratch_shapes=[pltpu.VMEM((B, D), jnp.float32)])(x, ohn, cntn,
                                                                 xv)


def _neadd(g0, g1):
    def body(a_ref, b_ref, o_ref):
        o_ref[...] = a_ref[...] + b_ref[...]

    return _PC(body, grid=(NEP // EBLK,),
               in_specs=[_bs((EBLK, D), lambda i: (i, 0)),
                         _bs((EBLK, D), lambda i: (i, 0))],
               out_specs=_bs((EBLK, D), lambda i: (i, 0)),
               out_shape=jax.ShapeDtypeStruct((NEP, D), jnp.float32))(g0, g1)


@functools.cache
def _mesh():
    return plsc.VectorSubcoreMesh(core_axis_name="c", subcore_axis_name="s")


def _sc_gather(table, idx3, wd, ct):
    nrows = NW * ct * 128

    @functools.partial(
        pl.kernel,
        out_type=jax.ShapeDtypeStruct((nrows, wd), jnp.float32),
        mesh=_mesh(),
        scratch_types=[pltpu.VMEM((ct, 128), jnp.int32),
                       pltpu.VMEM((128, wd), jnp.float32),
                       pltpu.SemaphoreType.DMA],
    )
    def k(tab_hbm, idx_hbm, out_hbm, idx_v, rows_v, sem):
        c = lax.axis_index("c")
        s = lax.axis_index("s")
        w = c * 16 + s
        pltpu.sync_copy(idx_hbm.at[w], idx_v)
        base = w * (ct * 128)

        def body(j, carry):
            pltpu.async_copy(tab_hbm.at[idx_v.at[j]], rows_v, sem).wait()
            pltpu.sync_copy(rows_v, out_hbm.at[pl.ds(base + j * 128, 128)])
            return carry

        lax.fori_loop(0, ct, body, 0, unroll=False)

    return k(table, idx3)


def _sc_gather2(ta, ia, wa, tb, ib, wb, ct):
    nrows = NW * ct * 128

    @functools.partial(
        pl.kernel,
        out_type=[jax.ShapeDtypeStruct((nrows, wa), jnp.float32),
                  jax.ShapeDtypeStruct((nrows, wb), jnp.float32)],
        mesh=_mesh(),
        scratch_types=[pltpu.VMEM((ct, 128), jnp.int32),
                       pltpu.VMEM((ct, 128), jnp.int32),
                       pltpu.VMEM((128, wa), jnp.float32),
                       pltpu.VMEM((128, wb), jnp.float32),
                       pltpu.SemaphoreType.DMA],
    )
    def k(ta_hbm, ia_hbm, tb_hbm, ib_hbm, oa_hbm, ob_hbm, ia_v, ib_v, ra_v,
          rb_v, sem):
        c = lax.axis_index("c")
        s = lax.axis_index("s")
        w = c * 16 + s
        pltpu.sync_copy(ia_hbm.at[w], ia_v)
        pltpu.sync_copy(ib_hbm.at[w], ib_v)
        base = w * (ct * 128)

        def body(j, carry):
            pltpu.async_copy(ta_hbm.at[ia_v.at[j]], ra_v, sem).wait()
            pltpu.sync_copy(ra_v, oa_hbm.at[pl.ds(base + j * 128, 128)])
            pltpu.async_copy(tb_hbm.at[ib_v.at[j]], rb_v, sem).wait()
            pltpu.sync_copy(rb_v, ob_hbm.at[pl.ds(base + j * 128, 128)])
            return carry

        lax.fori_loop(0, ct, body, 0, unroll=False)

    return k(ta, ia, tb, ib)


def _sc_scatter_edges(rows, idx3, wd, zeros):
    @functools.partial(
        pl.kernel,
        out_type=jax.ShapeDtypeStruct((2 * NnP, wd), jnp.float32),
        mesh=_mesh(),
        scratch_types=[pltpu.VMEM((CTW, 128), jnp.int32),
                       pltpu.VMEM((128, wd), jnp.float32),
                       pltpu.VMEM_SHARED((NnP, wd), jnp.float32)],
    )
    def k(rows_hbm, idx_hbm, z_hbm, out_hbm, idx_v, buf_v, shared):
        c = lax.axis_index("c")
        s = lax.axis_index("s")
        w = c * 16 + s
        pltpu.sync_copy(idx_hbm.at[w], idx_v)

        @pl.when(s == 0)
        def _():
            pltpu.sync_copy(z_hbm, shared)

        plsc.subcore_barrier()
        base = w * (CTW * 128)

        def body(j, carry):
            pltpu.sync_copy(rows_hbm.at[pl.ds(base + j * 128, 128)], buf_v)
            pltpu.sync_copy(buf_v, shared.at[idx_v.at[j]], add=True)
            return carry

        lax.fori_loop(0, CTW, body, 0, unroll=False)
        plsc.subcore_barrier()
        nper = NnP // 16
        pltpu.sync_copy(shared.at[pl.ds(s * nper, nper)],
                        out_hbm.at[pl.ds(c * NnP + s * nper, nper)])

    return k(rows, idx3, zeros)


def _sc_scatter_heads(rows2, idx3, zeros):
    wd = 128

    @functools.partial(
        pl.kernel,
        out_type=jax.ShapeDtypeStruct((2 * NnP, wd), jnp.float32),
        mesh=_mesh(),
        scratch_types=[pltpu.VMEM((CTS, 128), jnp.int32),
                       pltpu.VMEM((128, wd), jnp.float32),
                       pltpu.VMEM_SHARED((NnP, wd), jnp.float32)],
    )
    def k(rows_hbm, idx_hbm, z_hbm, out_hbm, idx_v, buf_v, shared):
        c = lax.axis_index("c")
        s = lax.axis_index("s")
        pltpu.sync_copy(idx_hbm.at[s], idx_v)

        @pl.when(s == 0)
        def _():
            pltpu.sync_copy(z_hbm, shared)

        plsc.subcore_barrier()
        base = c * EP + s * (CTS * 128)

        def body(j, carry):
            pltpu.sync_copy(rows_hbm.at[pl.ds(base + j * 128, 128)], buf_v)
            pltpu.sync_copy(buf_v, shared.at[idx_v.at[j]], add=True)
            return carry

        lax.fori_loop(0, CTS, body, 0, unroll=False)
        plsc.subcore_barrier()
        nper = NnP // 16
        pltpu.sync_copy(shared.at[pl.ds(s * nper, nper)],
                        out_hbm.at[pl.ds(c * NnP + s * nper, nper)])

    return k(rows2, idx3, zeros)


def _pad_idx(ix, total, shape):
    ix = ix.astype(jnp.int32)
    ix = jnp.pad(ix, (0, total - ix.shape[0]))
    return ix.reshape(shape)


def kernel(node_type, node_state_type, edge_index, edge_type, batch,
           non_edge_index, node_table, state_table, edge_table, virt_table,
           layer_params):
    f32 = jnp.float32
    batch = batch.astype(jnp.int32)
    src_w = _pad_idx(edge_index[0], EP, (NW, CTW, 128))
    dst_w = _pad_idx(edge_index[1], EP, (NW, CTW, 128))
    dst_s = _pad_idx(edge_index[1], EP, (16, CTS, 128))
    ne0_w = _pad_idx(non_edge_index[0], NEP, (NW, CTN, 128))
    ne1_w = _pad_idx(non_edge_index[1], NEP, (NW, CTN, 128))

    neg = jnp.full((NnP - Nn,), -1, jnp.int32)
    ids_aug = jnp.concatenate([batch, jnp.arange(B, dtype=jnp.int32), neg])
    oha = jax.nn.one_hot(ids_aug, B, dtype=f32)
    ids_n = jnp.concatenate([batch, jnp.full((NnP - N,), -1, jnp.int32)])
    ohn = jax.nn.one_hot(ids_n, B, dtype=f32)
    ids_v = jnp.concatenate([jnp.full((N,), -1, jnp.int32),
                             jnp.arange(B, dtype=jnp.int32), neg])
    ohv = jax.nn.one_hot(ids_v, B, dtype=f32)
    et_p = jnp.pad(edge_type.astype(jnp.int32), (0, EP - E),
                   constant_values=-1)
    ohet = jax.nn.one_hot(et_p, 8, dtype=f32)

    div = 1.0 / (10000.0 ** (jnp.arange(0, D, 2, dtype=f32) / D))
    pos = jnp.arange(4, dtype=f32)
    p4 = jnp.stack([jnp.sin(pos[:, None] * div[None, :]),
                    jnp.cos(pos[:, None] * div[None, :])],
                   axis=-1).reshape(4, D)
    tbl = jnp.zeros((32, D), f32)
    tbl = tbl.at[0:21].set(node_table)
    tbl = tbl.at[21].set(virt_table[0])
    tbl = tbl.at[22:26].set(state_table + p4)
    ids1 = jnp.concatenate([node_type.astype(jnp.int32),
                            jnp.full((B,), 21, jnp.int32), neg])
    ids2 = jnp.concatenate([jnp.clip(node_state_type.astype(jnp.int32), 0, 3)
                            + 22, jnp.full((B,), -1, jnp.int32), neg])
    ohe = jax.nn.one_hot(ids1, 32, dtype=f32) + jax.nn.one_hot(ids2, 32,
                                                               dtype=f32)
    x = _embed(ohe, tbl)

    cnts = _colsum(ohet)
    t4row = jnp.zeros((1, D), f32).at[0, 0].set(1.0)
    t5row = ((cnts[0:4, 0] @ edge_table)[None, :] + 2.0 * N * t4row) \
        / float(E + 2 * N)
    t6 = jnp.concatenate([edge_table, t4row, t5row,
                          jnp.zeros((2, D), f32)], axis=0)
    et8 = jnp.concatenate([edge_table, jnp.zeros((4, D), f32)], axis=0)
    cntn = _colsum(ohn)

    z128 = jnp.zeros((NnP, 128), f32)

    for p in layer_params:
        wcat = jnp.concatenate([p['Wq'], p['Wk'], p['Wv'], p['Wskip']],
                               axis=1)
        wa, wb = wcat[0:D], wcat[D:2 * D]
        bcat = jnp.concatenate([p['bq'], p['bk'], p['bv'], p['bskip']])[None]
        eet = jnp.concatenate(
            [t6[0:6] @ p['We'], jnp.zeros((2, 256), f32)], axis=0)

        st = _lnstats(x, oha)
        xn = _norm(x, oha, st)
        gs = _sc_gather(xn, src_w, 128, CTW)
        msgv = _msg(gs, ohet, et8)
        agg2 = _sc_scatter_edges(msgv, dst_w, 128, z128)
        uv = _uvsum(xn, ohn, t4row)
        kvirt = lax.dynamic_slice_in_dim(xn, N, B)
        q, kv, sk = _aggqkvs(agg2[:NnP], agg2[NnP:], xn, ohn, ohv, uv, kvirt,
                             t4row, t5row, p['gen_W'], p['gen_b'][None], wa,
                             wb, bcat)
        vq = lax.dynamic_slice_in_dim(q, N, B)
        vkv = lax.dynamic_slice_in_dim(kv, N, B)
        sa, bnd_s = _structalpha(q, kv, ohn, vq, vkv, eet)
        gq, gkv = _sc_gather2(q, dst_w, 256, kv, src_w, 512, CTW)
        ae, bnd_e = _alpha(gq, gkv, ohet, eet)
        buk = _bucket(ae, ohet, bnd_e, bnd_s)
        cnt2 = _sc_scatter_edges(buk, dst_w, 128, z128)
        cu = _cntu(sa, ohn, bnd_e, bnd_s)
        cmx = _cmax(cnt2[:NnP], cnt2[NnP:], ohv, ohn, cu, sa, bnd_e, bnd_s)
        cv = lax.dynamic_slice_in_dim(cmx, N, B)
        nums, dens, uvn, uvd = _structnum(cmx, sa, kv, ohn, cv, vkv, eet)
        cg = _sc_gather(cmx, dst_w, 128, CTW)
        r0, r1, denr = _rbuild(ae, gkv, ohet, cg, eet)
        nd = _sc_scatter_heads(jnp.concatenate([r0, r1], axis=0), dst_s, z128)
        ndd = _sc_scatter_edges(denr, dst_w, 128, z128)
        lh = _outk(nd[:NnP], nd[NnP:], ndd[:NnP], ndd[NnP:], nums, dens, uvn,
                   uvd, ohv, sk, p['lin_W'], p['lin_b'][None])
        st2 = _lnstats(lh, oha)
        x = _ffn(lh, oha, st2, x, p['ff_W1'], p['ff_b1'][None], p['ff_W2'],
                 p['ff_b2'][None])

    glob = _pool(x, ohn, cntn, lax.dynamic_slice_in_dim(x, N, B))
    g0, g1 = _sc_gather2(x, ne0_w, 128, x, ne1_w, 128, CTN)
    nes = _neadd(g0, g1)
    return jnp.concatenate([x[:N], glob, nes[:NE]], axis=0)
